# baseline (speedup 1.0000x reference)
import functools

import jax
import jax.numpy as jnp
from jax import lax
from jax.experimental import pallas as pl
from jax.experimental.pallas import tpu as pltpu
from jax.experimental.pallas import tpu_sc as plsc

L16 = 16
PADW = 128


def _make_sc_kernel(B, L, D, V):
    info = plsc.get_sparse_core_info()
    NC, NS = info.num_cores, info.num_subcores
    NW = NC * NS
    BBLK = B // NW
    assert BBLK == 128 and L % 2 == 0 and D % L16 == 0
    NBV = BBLK // L16
    CG = 8

    mesh = plsc.VectorSubcoreMesh(core_axis_name="c", subcore_axis_name="s")

    @functools.partial(
        pl.kernel,
        mesh=mesh,
        compiler_params=pltpu.CompilerParams(
            use_tc_tiling_on_sc=True, needs_layout_passes=False
        ),
        out_type=jax.ShapeDtypeStruct((L, D, B), jnp.float32),
        scratch_types=[
            pltpu.VMEM((L, PADW), jnp.float32),
            pltpu.VMEM((L, BBLK), jnp.int32),
            pltpu.VMEM((BBLK, PADW), jnp.float32),
            pltpu.VMEM((BBLK, PADW), jnp.float32),
            pltpu.VMEM((D, BBLK + 5), jnp.float32),
            pltpu.VMEM((D, BBLK + 5), jnp.float32),
            pltpu.SemaphoreType.DMA,
            pltpu.SemaphoreType.DMA,
            pltpu.SemaphoreType.DMA,
            pltpu.SemaphoreType.DMA,
        ],
    )
    def sc_kernel(idx_hbm, word_hbm, pos_hbm, out_hbm,
                  pos_v, idx_v, rows0, rows1, t0, t1,
                  gsem0, gsem1, osem0, osem1):
        rows = (rows0, rows1)
        tb = (t0, t1)
        gsem = (gsem0, gsem1)
        osem = (osem0, osem1)
        wid = lax.axis_index("s") * NC + lax.axis_index("c")
        b0 = wid * BBLK
        pltpu.sync_copy(pos_hbm, pos_v)
        pltpu.sync_copy(idx_hbm.at[:, pl.ds(b0, BBLK)], idx_v)

        def issue_gather(l, p):
            pltpu.async_copy(word_hbm.at[idx_v.at[l]], rows[p], gsem[p])

        def wait_gather(p):
            pltpu.make_async_copy(
                word_hbm.at[pl.ds(0, BBLK)], rows[p], gsem[p]
            ).wait()

        def wait_writeback(p):
            pltpu.make_async_copy(
                tb[p].at[:, pl.ds(0, BBLK)],
                out_hbm.at[0, :, pl.ds(0, BBLK)],
                osem[p],
            ).wait()

        def transpose_add_flush(l, p):
            rowv = [
                jnp.broadcast_to(jnp.int32(g * L16), (L16,))
                + lax.iota(jnp.int32, L16)
                for g in range(D // L16)
            ]
            pbs = [pos_v[l, pl.ds(g * L16, L16)] for g in range(D // L16)]
            UN = 8

            def b_body(bb, carry2):
                for u in range(UN):
                    b = bb * UN + u
                    bv = jnp.broadcast_to(b, (L16,))
                    vs = [
                        rows[p][b, pl.ds(g * L16, L16)] + pbs[g]
                        for g in range(D // L16)
                    ]
                    for g in range(D // L16):
                        plsc.store_scatter(tb[p], [rowv[g], bv], vs[g])
                return carry2

            lax.fori_loop(0, BBLK // UN, b_body, 0)
            pltpu.async_copy(
                tb[p].at[:, pl.ds(0, BBLK)],
                out_hbm.at[l, :, pl.ds(b0, BBLK)],
                osem[p],
            )

        issue_gather(0, 0)

        def loop_body(j, carry):
            a = 2 * j
            @pl.when(j > 0)
            def _():
                wait_writeback(1)
            issue_gather(a + 1, 1)
            wait_gather(0)
            transpose_add_flush(a, 0)
            @pl.when(j < L // 2 - 1)
            def _():
                wait_writeback(0)
                issue_gather(a + 2, 0)
            wait_gather(1)
            transpose_add_flush(a + 1, 1)
            return carry

        lax.fori_loop(0, L // 2, loop_body, 0)
        wait_writeback(0)
        wait_writeback(1)

    return sc_kernel


def kernel(inputs, word_table, pos_table):
    B, L = inputs.shape
    V, D = word_table.shape
    idx_t = jnp.transpose(inputs).astype(jnp.int32)
    wpad = jnp.concatenate(
        [word_table, jnp.zeros((V, PADW - D), jnp.float32)], axis=1
    )
    ppad = jnp.concatenate(
        [pos_table, jnp.zeros((L, PADW - D), jnp.float32)], axis=1
    )
    sc = _make_sc_kernel(B, L, D, V)
    out_t = sc(idx_t, wpad, ppad)
    return jnp.transpose(out_t, (2, 0, 1))


# speedup vs baseline: 1.3162x; 1.3162x over previous
import functools

import jax
import jax.numpy as jnp
from jax import lax
from jax.experimental import pallas as pl
from jax.experimental.pallas import tpu as pltpu
from jax.experimental.pallas import tpu_sc as plsc

L16 = 16
PADW = 128


def _make_sc_kernel(B, L, D, V):
    info = plsc.get_sparse_core_info()
    NC, NS = info.num_cores, info.num_subcores
    NW = NC * NS
    BBLK = B // NW
    assert BBLK == 128 and L % 2 == 0 and D % L16 == 0
    NBV = BBLK // L16
    CG = 8

    mesh = plsc.VectorSubcoreMesh(core_axis_name="c", subcore_axis_name="s")

    @functools.partial(
        pl.kernel,
        mesh=mesh,
        compiler_params=pltpu.CompilerParams(
            use_tc_tiling_on_sc=True, needs_layout_passes=False
        ),
        out_type=jax.ShapeDtypeStruct((L, D, B), jnp.float32),
        scratch_types=[
            pltpu.VMEM((L, PADW), jnp.float32),
            pltpu.VMEM((8, BBLK), jnp.int32),
            pltpu.VMEM((8, BBLK), jnp.int32),
            pltpu.VMEM((BBLK, PADW + 5), jnp.float32),
            pltpu.VMEM((BBLK, PADW + 5), jnp.float32),
            pltpu.VMEM((D, BBLK), jnp.float32),
            pltpu.VMEM((D, BBLK), jnp.float32),
            pltpu.SemaphoreType.DMA,
            pltpu.SemaphoreType.DMA,
            pltpu.SemaphoreType.DMA,
            pltpu.SemaphoreType.DMA,
            pltpu.SemaphoreType.DMA,
            pltpu.SemaphoreType.DMA,
        ],
    )
    def sc_kernel(idx_hbm, word_hbm, pos_hbm, out_hbm,
                  pos_v, idxt0, idxt1, rows0, rows1, t0, t1,
                  gsem0, gsem1, osem0, osem1, isem0, isem1):
        rows = (rows0, rows1)
        tb = (t0, t1)
        idxt = (idxt0, idxt1)
        gsem = (gsem0, gsem1)
        osem = (osem0, osem1)
        isem = (isem0, isem1)
        wid = lax.axis_index("s") * NC + lax.axis_index("c")
        b0 = wid * BBLK
        pltpu.sync_copy(pos_hbm, pos_v)

        def issue_idx(lb, q):
            pltpu.async_copy(
                idx_hbm.at[pl.ds(lb * 8, 8), pl.ds(b0, BBLK)], idxt[q],
                isem[q],
            )

        def wait_idx(q):
            pltpu.make_async_copy(
                idx_hbm.at[pl.ds(0, 8), pl.ds(0, BBLK)], idxt[q], isem[q]
            ).wait()

        def with_buf(qval, fn):
            @pl.when(qval == 0)
            def _():
                fn(0)

            @pl.when(qval == 1)
            def _():
                fn(1)

        def issue_gather(l, p):
            def go(q):
                pltpu.async_copy(
                    word_hbm.at[idxt[q].at[l % 8]],
                    rows[p].at[:, pl.ds(0, PADW)],
                    gsem[p],
                )

            with_buf((l // 8) % 2, go)

        def wait_gather(p):
            pltpu.make_async_copy(
                word_hbm.at[pl.ds(0, BBLK)],
                rows[p].at[:, pl.ds(0, PADW)],
                gsem[p],
            ).wait()

        def wait_writeback(p):
            pltpu.make_async_copy(
                tb[p], out_hbm.at[0, :, pl.ds(0, BBLK)], osem[p]
            ).wait()

        def transpose_add_flush(l, p):
            rowidx = [
                jnp.broadcast_to(jnp.int32(k * L16), (L16,))
                + lax.iota(jnp.int32, L16)
                for k in range(NBV)
            ]
            pidx = jnp.broadcast_to(l, (L16,))

            def cg_body(cg, carry2):
                cs = [cg * CG + cc for cc in range(CG)]
                cvec = [jnp.broadcast_to(c, (L16,)) for c in cs]
                pb = [
                    plsc.load_gather(pos_v, [pidx, cvec[cc]])
                    for cc in range(CG)
                ]
                for k in range(NBV):
                    vs = [
                        plsc.load_gather(rows[p], [rowidx[k], cvec[cc]])
                        for cc in range(CG)
                    ]
                    for cc in range(CG):
                        tb[p][cs[cc], pl.ds(k * L16, L16)] = vs[cc] + pb[cc]
                return carry2

            lax.fori_loop(0, D // CG, cg_body, 0)
            pltpu.async_copy(
                tb[p], out_hbm.at[l, :, pl.ds(b0, BBLK)], osem[p]
            )

        issue_idx(0, 0)
        wait_idx(0)
        issue_gather(0, 0)
        issue_idx(1, 1)

        def loop_body(j, carry):
            a = 2 * j
            @pl.when(jnp.logical_and(a % 8 == 0,
                                     jnp.logical_and(a > 0, a + 8 < L)))
            def _():
                lb1 = a // 8 + 1
                with_buf(lb1 % 2, lambda q: issue_idx(lb1, q))

            @pl.when(j > 0)
            def _():
                wait_writeback(1)
            issue_gather(a + 1, 1)
            wait_gather(0)
            transpose_add_flush(a, 0)
            @pl.when(j < L // 2 - 1)
            def _():
                wait_writeback(0)
                @pl.when((a + 2) % 8 == 0)
                def _():
                    with_buf(((a + 2) // 8) % 2, wait_idx)
                issue_gather(a + 2, 0)
            wait_gather(1)
            transpose_add_flush(a + 1, 1)
            return carry

        lax.fori_loop(0, L // 2, loop_body, 0)
        wait_writeback(0)
        wait_writeback(1)

    return sc_kernel


def kernel(inputs, word_table, pos_table):
    B, L = inputs.shape
    V, D = word_table.shape
    idx_t = jnp.transpose(inputs).astype(jnp.int32)
    wpad = jnp.concatenate(
        [word_table, jnp.zeros((V, PADW - D), jnp.float32)], axis=1
    )
    ppad = jnp.concatenate(
        [pos_table, jnp.zeros((L, PADW - D), jnp.float32)], axis=1
    )
    sc = _make_sc_kernel(B, L, D, V)
    out_t = sc(idx_t, wpad, ppad)
    return jnp.transpose(out_t, (2, 0, 1))


# speedup vs baseline: 1.6526x; 1.2555x over previous
import functools

import jax
import jax.numpy as jnp
from jax import lax
from jax.experimental import pallas as pl
from jax.experimental.pallas import tpu as pltpu
from jax.experimental.pallas import tpu_sc as plsc

L16 = 16


def _make_sc_kernel(B, L, D, V):
    info = plsc.get_sparse_core_info()
    NC, NS = info.num_cores, info.num_subcores
    NW = NC * NS
    assert B % NW == 0
    ROWS_PER_W = B // NW
    CB = 2
    NCHUNK = ROWS_PER_W // CB
    CROWS = CB * L
    WROWS = ROWS_PER_W * L
    SUBS = [128] * (CROWS // 128)
    if CROWS % 128:
        SUBS.append(CROWS % 128)

    mesh = plsc.VectorSubcoreMesh(core_axis_name="c", subcore_axis_name="s")

    @functools.partial(
        pl.kernel,
        mesh=mesh,
        compiler_params=pltpu.CompilerParams(use_tc_tiling_on_sc=False),
        out_type=jax.ShapeDtypeStruct((B, L, D), jnp.float32),
        scratch_types=[
            pltpu.VMEM((L, D), jnp.float32),
            pltpu.VMEM((WROWS,), jnp.int32),
            pltpu.VMEM((CROWS, D), jnp.float32),
            pltpu.VMEM((CROWS, D), jnp.float32),
            pltpu.SemaphoreType.DMA,
            pltpu.SemaphoreType.DMA,
            pltpu.SemaphoreType.DMA,
            pltpu.SemaphoreType.DMA,
        ],
    )
    def sc_kernel(idx_hbm, word_hbm, pos_hbm, out_hbm,
                  pos_v, idx_v, rows0, rows1, gsem0, gsem1, osem0, osem1):
        rows = (rows0, rows1)
        gsem = (gsem0, gsem1)
        osem = (osem0, osem1)
        wid = lax.axis_index("s") * NC + lax.axis_index("c")
        wbase = wid * WROWS
        pltpu.sync_copy(pos_hbm, pos_v)
        pltpu.sync_copy(idx_hbm.at[pl.ds(wbase, WROWS)], idx_v)

        def issue_gathers(g, p):
            off = 0
            for sz in SUBS:
                pltpu.async_copy(
                    word_hbm.at[idx_v.at[pl.ds(g * CROWS + off, sz)]],
                    rows[p].at[pl.ds(off, sz)],
                    gsem[p],
                )
                off += sz

        def wait_gathers(p):
            pltpu.make_async_copy(
                word_hbm.at[pl.ds(0, CROWS)], rows[p], gsem[p]
            ).wait()

        def wait_writeback(p):
            for rb in range(CB):
                pltpu.make_async_copy(
                    rows[p].at[pl.ds(rb * L, L)], out_hbm.at[0], osem[p]
                ).wait()

        def add_and_flush(g, p):
            def add_body(l, carry2):
                for c in range(D // L16):
                    pv = pos_v[l, pl.ds(c * L16, L16)]
                    for rb in range(CB):
                        plsc.addupdate(
                            rows[p].at[rb * L + l, pl.ds(c * L16, L16)], pv
                        )
                return carry2

            lax.fori_loop(0, L, add_body, 0)
            bb = wid * ROWS_PER_W + g * CB
            for rb in range(CB):
                pltpu.async_copy(
                    rows[p].at[pl.ds(rb * L, L)], out_hbm.at[bb + rb], osem[p]
                )

        issue_gathers(0, 0)

        def loop_body(j, carry):
            a = 2 * j
            @pl.when(j > 0)
            def _():
                wait_writeback(1)
            issue_gathers(a + 1, 1)
            wait_gathers(0)
            add_and_flush(a, 0)
            @pl.when(j < NCHUNK // 2 - 1)
            def _():
                wait_writeback(0)
                issue_gathers(a + 2, 0)
            wait_gathers(1)
            add_and_flush(a + 1, 1)
            return carry

        lax.fori_loop(0, NCHUNK // 2, loop_body, 0)
        wait_writeback(0)
        wait_writeback(1)

    return sc_kernel


def kernel(inputs, word_table, pos_table):
    B, L = inputs.shape
    V, D = word_table.shape
    idx = inputs.reshape(B * L).astype(jnp.int32)
    sc = _make_sc_kernel(B, L, D, V)
    return sc(idx, word_table, pos_table)
